# Initial kernel scaffold; baseline (speedup 1.0000x reference)
import jax
import jax.numpy as jnp
from jax.experimental import pallas as pl


def kernel(inputs, token_table, position_table):
    raise NotImplementedError("write your pallas kernel here")


# speedup vs baseline: 1.3906x; 1.3906x over previous
import functools

import jax
import jax.numpy as jnp
from jax import lax
from jax.experimental import pallas as pl
from jax.experimental.pallas import tpu as pltpu
from jax.experimental.pallas import tpu_sc as plsc

SEQ_LEN = 200
EMBED = 32
BATCH = 4096
LANES = 16

NC, NS = 2, 16
NW = NC * NS
ROWS_TOTAL = BATCH * SEQ_LEN
GATHER = 100
SEQ_PER_CHUNK = 4
CHUNK = SEQ_PER_CHUNK * SEQ_LEN
GPC = CHUNK // GATHER
ROWS_PER_W = ROWS_TOTAL // NW
CHUNKS_PER_W = ROWS_PER_W // CHUNK
IDX_ROWS_PER_CHUNK = CHUNK // GATHER

_mesh = plsc.VectorSubcoreMesh(core_axis_name="c", subcore_axis_name="s")


@functools.partial(
    pl.kernel,
    out_type=jax.ShapeDtypeStruct((ROWS_TOTAL, EMBED), jnp.float32),
    mesh=_mesh,
    scratch_types=[
        pltpu.VMEM((GPC, GATHER), jnp.int32),
        pltpu.VMEM((CHUNK, EMBED), jnp.float32),
        pltpu.VMEM((SEQ_LEN, EMBED), jnp.float32),
        pltpu.SemaphoreType.DMA,
    ],
    compiler_params=pltpu.CompilerParams(use_tc_tiling_on_sc=False),
)
def _emb_kernel(idx_hbm, tok_hbm, pos_hbm, out_hbm, idx_v, rows_v, pos_v, sem):
    wid = lax.axis_index("s") * NC + lax.axis_index("c")
    pltpu.sync_copy(pos_hbm, pos_v)

    def chunk_body(c, carry):
        irow0 = wid * (ROWS_PER_W // GATHER) + c * IDX_ROWS_PER_CHUNK
        row0 = wid * ROWS_PER_W + c * CHUNK
        pltpu.sync_copy(idx_hbm.at[pl.ds(irow0, IDX_ROWS_PER_CHUNK)], idx_v)
        descs = [
            pltpu.async_copy(
                tok_hbm.at[idx_v.at[k]],
                rows_v.at[pl.ds(k * GATHER, GATHER)],
                sem,
            )
            for k in range(GPC)
        ]
        for d in descs:
            d.wait()

        def add_body(j, carry2):
            p0 = pos_v[j, pl.ds(0, LANES)]
            p1 = pos_v[j, pl.ds(LANES, LANES)]
            for s in range(SEQ_PER_CHUNK):
                r = s * SEQ_LEN + j
                rows_v[r, pl.ds(0, LANES)] = rows_v[r, pl.ds(0, LANES)] + p0
                rows_v[r, pl.ds(LANES, LANES)] = (
                    rows_v[r, pl.ds(LANES, LANES)] + p1
                )
            return carry2

        lax.fori_loop(0, SEQ_LEN, add_body, 0)
        pltpu.sync_copy(rows_v, out_hbm.at[pl.ds(row0, CHUNK)])
        return carry

    lax.fori_loop(0, CHUNKS_PER_W, chunk_body, 0)


def kernel(inputs, token_table, position_table):
    idx2d = inputs.astype(jnp.int32).reshape(ROWS_TOTAL // GATHER, GATHER)
    out = _emb_kernel(idx2d, token_table, position_table)
    return out.reshape(BATCH, SEQ_LEN, EMBED)


# speedup vs baseline: 1.4851x; 1.0680x over previous
import functools

import jax
import jax.numpy as jnp
from jax import lax
from jax.experimental import pallas as pl
from jax.experimental.pallas import tpu as pltpu
from jax.experimental.pallas import tpu_sc as plsc

SEQ_LEN = 200
EMBED = 32
BATCH = 4096
LANES = 16

NC, NS = 2, 16
NW = NC * NS
ROWS_TOTAL = BATCH * SEQ_LEN
GATHER = 100
SEQ_PER_CHUNK = 4
CHUNK = SEQ_PER_CHUNK * SEQ_LEN
GPC = CHUNK // GATHER
ROWS_PER_W = ROWS_TOTAL // NW
CHUNKS_PER_W = ROWS_PER_W // CHUNK
IDX_ROWS_PER_W = ROWS_PER_W // GATHER

_mesh = plsc.VectorSubcoreMesh(core_axis_name="c", subcore_axis_name="s")


@functools.partial(
    pl.kernel,
    out_type=jax.ShapeDtypeStruct((ROWS_TOTAL, EMBED), jnp.float32),
    mesh=_mesh,
    scratch_types=[
        pltpu.VMEM((IDX_ROWS_PER_W, GATHER), jnp.int32),
        pltpu.VMEM((CHUNK, EMBED), jnp.float32),
        pltpu.VMEM((CHUNK, EMBED), jnp.float32),
        pltpu.VMEM((SEQ_LEN, EMBED), jnp.float32),
        pltpu.SemaphoreType.DMA,
        pltpu.SemaphoreType.DMA,
        pltpu.SemaphoreType.DMA,
        pltpu.SemaphoreType.DMA,
    ],
    compiler_params=pltpu.CompilerParams(use_tc_tiling_on_sc=False),
)
def _emb_kernel(idx_hbm, tok_hbm, pos_hbm, out_hbm,
                idx_v, rows0, rows1, pos_v, g0, g1, o0, o1):
    wid = lax.axis_index("s") * NC + lax.axis_index("c")
    pltpu.sync_copy(pos_hbm, pos_v)
    pltpu.sync_copy(idx_hbm.at[pl.ds(wid * IDX_ROWS_PER_W, IDX_ROWS_PER_W)],
                    idx_v)
    base_row = wid * ROWS_PER_W

    rows = (rows0, rows1)
    gsem = (g0, g1)
    osem = (o0, o1)

    def issue(c, b):
        for k in range(GPC):
            pltpu.async_copy(
                tok_hbm.at[idx_v.at[c * GPC + k]],
                rows[b].at[pl.ds(k * GATHER, GATHER)],
                gsem[b],
            )

    def complete(c, b):
        pltpu.make_async_copy(
            tok_hbm.at[pl.ds(0, CHUNK)], rows[b], gsem[b]).wait()

        @pl.loop(0, SEQ_LEN)
        def _(j):
            p0 = pos_v[j, pl.ds(0, LANES)]
            p1 = pos_v[j, pl.ds(LANES, LANES)]
            for s in range(SEQ_PER_CHUNK):
                plsc.addupdate(rows[b].at[s * SEQ_LEN + j, pl.ds(0, LANES)],
                               p0)
                plsc.addupdate(
                    rows[b].at[s * SEQ_LEN + j, pl.ds(LANES, LANES)], p1)

        pltpu.async_copy(
            rows[b], out_hbm.at[pl.ds(base_row + c * CHUNK, CHUNK)], osem[b])

    @pl.loop(0, CHUNKS_PER_W // 2)
    def _(g):
        for b in range(2):
            @pl.when(g > 0)
            def _():
                pltpu.make_async_copy(
                    rows[b], out_hbm.at[pl.ds(0, CHUNK)], osem[b]).wait()
            issue(2 * g + b, b)
        for b in range(2):
            complete(2 * g + b, b)

    for b in range(2):
        pltpu.make_async_copy(
            rows[b], out_hbm.at[pl.ds(0, CHUNK)], osem[b]).wait()


def kernel(inputs, token_table, position_table):
    idx2d = inputs.astype(jnp.int32).reshape(ROWS_TOTAL // GATHER, GATHER)
    out = _emb_kernel(idx2d, token_table, position_table)
    return out.reshape(BATCH, SEQ_LEN, EMBED)
